# Initial kernel scaffold; baseline (speedup 1.0000x reference)
import jax
import jax.numpy as jnp
from jax.experimental import pallas as pl


def kernel(class_logits, box_regression, priors):
    raise NotImplementedError("write your pallas kernel here")


# speedup vs baseline: 4.7452x; 4.7452x over previous
import functools

import jax
import jax.numpy as jnp
from jax import lax
from jax.experimental import pallas as pl

N = 20000
C = 81
MAX_DET = 100
NMS_THRESHOLD = 0.6
SCORE_THRESHOLD = 0.3

NPAD = 20480
ROWS = NPAD // 128
CB = 16


def _decode_body(dx, dy, dw, dh, px, py, pw, ph, x0, y0, x1, y1, area):
    cx = dx[...] * pw[...] + px[...]
    cy = dy[...] * ph[...] + py[...]
    w = jnp.exp(dw[...]) * pw[...]
    h = jnp.exp(dh[...]) * ph[...]
    x0[...] = cx - w / 2.0
    y0[...] = cy - h / 2.0
    x1[...] = cx + w / 2.0
    y1[...] = cy + h / 2.0
    area[...] = jnp.maximum(x1[...] - x0[...], 0.0) * jnp.maximum(
        y1[...] - y0[...], 0.0)


def _nms_body(logit_ref, x0r, y0r, x1r, y1r, arear, boxes_out, scores_out):
    p = jax.nn.sigmoid(logit_ref[...])
    p = jnp.where(p > SCORE_THRESHOLD, p, -1.0)
    x0 = x0r[...]
    y0 = y0r[...]
    x1 = x1r[...]
    y1 = y1r[...]
    area = arear[...]
    rio = lax.broadcasted_iota(jnp.int32, (ROWS, 128), 0)
    lio = lax.broadcasted_iota(jnp.int32, (ROWS, 128), 1)
    fio = rio * 128 + lio
    lane3 = lax.broadcasted_iota(jnp.int32, (CB, 1, 128), 2)

    def step(i, p):
        best = jnp.max(p, axis=(1, 2), keepdims=True)
        idx = jnp.min(jnp.where(p == best, fio, jnp.int32(1 << 30)),
                      axis=(1, 2), keepdims=True)
        selm = fio[None] == idx
        selmf = selm.astype(jnp.float32)
        bx0 = jnp.sum(x0 * selmf, axis=(1, 2), keepdims=True)
        by0 = jnp.sum(y0 * selmf, axis=(1, 2), keepdims=True)
        bx1 = jnp.sum(x1 * selmf, axis=(1, 2), keepdims=True)
        by1 = jnp.sum(y1 * selmf, axis=(1, 2), keepdims=True)
        a1 = jnp.maximum(bx1 - bx0, 0.0) * jnp.maximum(by1 - by0, 0.0)
        iw = jnp.maximum(jnp.minimum(bx1, x1) - jnp.maximum(bx0, x0), 0.0)
        ih = jnp.maximum(jnp.minimum(by1, y1) - jnp.maximum(by0, y0), 0.0)
        inter = iw * ih
        iou = inter / (a1 + area - inter + 1e-9)
        keep = best > 0.0
        sc = jnp.where(keep, best, 0.0)
        kb0 = jnp.where(keep, bx0, 0.0)
        kb1 = jnp.where(keep, by0, 0.0)
        kb2 = jnp.where(keep, bx1, 0.0)
        kb3 = jnp.where(keep, by1, 0.0)
        boxrow = jnp.where(
            lane3 == 0, kb0,
            jnp.where(lane3 == 1, kb1,
                      jnp.where(lane3 == 2, kb2,
                                jnp.where(lane3 == 3, kb3, 0.0))))
        boxes_out[:, pl.ds(i, 1), :] = boxrow
        scores_out[:, pl.ds(i, 1), :] = jnp.broadcast_to(sc, (CB, 1, 8))
        return jnp.where((iou > NMS_THRESHOLD) | selm, -1.0, p)

    lax.fori_loop(0, MAX_DET, step, p)


def _pad_col(a):
    return jnp.pad(a, (0, NPAD - N)).reshape(ROWS, 128)


@jax.jit
def kernel(class_logits, box_regression, priors):
    logits = class_logits[0]
    reg = box_regression[0]

    wx, wy, ww, wh = 10.0, 10.0, 5.0, 5.0
    cols = [
        _pad_col(reg[:, 0] / wx), _pad_col(reg[:, 1] / wy),
        _pad_col(reg[:, 2] / ww), _pad_col(reg[:, 3] / wh),
        _pad_col(priors[:, 0]), _pad_col(priors[:, 1]),
        _pad_col(priors[:, 2]), _pad_col(priors[:, 3]),
    ]

    x0, y0, x1, y1, area = pl.pallas_call(
        _decode_body,
        out_shape=[jax.ShapeDtypeStruct((ROWS, 128), jnp.float32)] * 5,
    )(*cols)

    logits_t = jnp.pad(logits.T[1:C], ((0, 0), (0, NPAD - N)),
                       constant_values=-1e9).reshape(C - 1, ROWS, 128)

    fullmap = lambda g: (0, 0)
    boxes_pad, scores_pad = pl.pallas_call(
        _nms_body,
        grid=((C - 1) // CB,),
        in_specs=[
            pl.BlockSpec((CB, ROWS, 128), lambda g: (g, 0, 0)),
            pl.BlockSpec((ROWS, 128), fullmap),
            pl.BlockSpec((ROWS, 128), fullmap),
            pl.BlockSpec((ROWS, 128), fullmap),
            pl.BlockSpec((ROWS, 128), fullmap),
            pl.BlockSpec((ROWS, 128), fullmap),
        ],
        out_specs=[
            pl.BlockSpec((CB, MAX_DET + 4, 128), lambda g: (g, 0, 0)),
            pl.BlockSpec((CB, MAX_DET + 4, 8), lambda g: (g, 0, 0)),
        ],
        out_shape=[
            jax.ShapeDtypeStruct((C - 1, MAX_DET + 4, 128), jnp.float32),
            jax.ShapeDtypeStruct((C - 1, MAX_DET + 4, 8), jnp.float32),
        ],
    )(logits_t, x0, y0, x1, y1, area)

    sel_boxes = boxes_pad[:, :MAX_DET, :4]
    sel_scores = scores_pad[:, :MAX_DET, 0]
    labels = jnp.broadcast_to(
        jnp.arange(1, C, dtype=jnp.int64)[:, None], (C - 1, MAX_DET))
    return sel_boxes, labels, sel_scores
